# baseline (speedup 1.0000x reference)
import functools

import jax
import jax.numpy as jnp
from jax import lax
from jax.experimental import pallas as pl
from jax.experimental.pallas import tpu as pltpu
from jax.experimental.pallas import tpu_sc as plsc

IN_NODES = 100000
OUT = 32
F = 16
E = IN_NODES * OUT
NC = 2
NS = 16
NW = NC * NS
NPW = IN_NODES // NW
NODE_F32 = OUT * F

_mesh = plsc.VectorSubcoreMesh(core_axis_name="c", subcore_axis_name="s")
_params = pltpu.CompilerParams(needs_layout_passes=False)


def _wid():
    return lax.axis_index("s") * NC + lax.axis_index("c")


def _squash(s):
    sq = jnp.sum(s**2, axis=1, keepdims=True)
    return sq / (1.0 + sq) * (s / jnp.sqrt(sq))


def _chunk_src(u_hbm, base, ci, chunk):
    return u_hbm.at[pl.ds(base + ci * (chunk * NODE_F32), chunk * NODE_F32)]


def _double_buffered(u_hbm, base, chunk, nchunk, buf0, buf1, sem0, sem1,
                     compute_chunk, init_carry):
    npairs = nchunk // 2

    pltpu.async_copy(_chunk_src(u_hbm, base, 0, chunk), buf0, sem0)

    def pair_body(i, carry):
        ci0 = 2 * i
        pltpu.async_copy(_chunk_src(u_hbm, base, ci0 + 1, chunk), buf1, sem1)
        pltpu.make_async_copy(_chunk_src(u_hbm, base, ci0, chunk), buf0, sem0).wait()
        carry = compute_chunk(buf0, ci0, carry)
        pltpu.async_copy(_chunk_src(u_hbm, base, ci0 + 2, chunk), buf0, sem0)
        pltpu.make_async_copy(
            _chunk_src(u_hbm, base, ci0 + 1, chunk), buf1, sem1
        ).wait()
        return compute_chunk(buf1, ci0 + 1, carry)

    carry = lax.fori_loop(0, npairs, pair_body, init_carry)
    pltpu.make_async_copy(
        _chunk_src(u_hbm, base, nchunk - 1, chunk), buf0, sem0
    ).wait()
    return compute_chunk(buf0, nchunk - 1, carry)


_SUM_CHUNK = 125


@functools.partial(
    pl.kernel,
    out_type=jax.ShapeDtypeStruct((NW, NODE_F32), jnp.float32),
    mesh=_mesh,
    compiler_params=_params,
    scratch_types=[
        pltpu.VMEM((_SUM_CHUNK * NODE_F32,), jnp.float32),
        pltpu.VMEM((_SUM_CHUNK * NODE_F32,), jnp.float32),
        pltpu.VMEM((NODE_F32,), jnp.float32),
        pltpu.SemaphoreType.DMA,
        pltpu.SemaphoreType.DMA,
    ],
)
def _pass_sum(u_hbm, out_hbm, buf0, buf1, obuf, sem0, sem1):
    wid = _wid()
    base = wid * (NPW * NODE_F32)

    def compute_chunk(buf, ci, accs):
        def node_body(ui, accs):
            nb = ui * NODE_F32
            return tuple(accs[o] + buf[pl.ds(nb + o * F, F)] for o in range(OUT))

        return lax.fori_loop(0, _SUM_CHUNK, node_body, accs)

    accs = _double_buffered(
        u_hbm, base, _SUM_CHUNK, NPW // _SUM_CHUNK, buf0, buf1, sem0, sem1,
        compute_chunk, tuple(jnp.zeros((F,), jnp.float32) for _ in range(OUT)),
    )
    for o in range(OUT):
        obuf[pl.ds(o * F, F)] = accs[o]
    pltpu.sync_copy(obuf, out_hbm.at[wid])


def _tree_sum(ps):
    while len(ps) > 1:
        ps = [a + b for a, b in zip(ps[::2], ps[1::2])] + (
            [ps[-1]] if len(ps) % 2 else []
        )
    return ps[0]


def _lane_shuffle(v, idx):
    return lax.gather(
        v,
        idx[:, None],
        lax.GatherDimensionNumbers(
            offset_dims=(), collapsed_slice_dims=(0,), start_index_map=(0,)
        ),
        (1,),
        mode=lax.GatherScatterMode.PROMISE_IN_BOUNDS,
    )


def _lane_sum_all(v):
    for s in (1, 2, 4, 8):
        idx = jnp.arange(16, dtype=jnp.int32) ^ s
        v = v + _lane_shuffle(v, idx)
    return v


def _group_logits(buf, vtb, bases, stride):
    prods = [[[] for _ in range(2)] for _ in bases]
    gs = [[[] for _ in range(2)] for _ in bases]
    for h in range(2):
        for f in range(F):
            vt = vtb[pl.ds(f * 32 + h * 16, 16)]
            for k, nb in enumerate(bases):
                g = plsc.load_gather(buf, [stride + (nb + h * 256 + f)])
                gs[k][h].append(g)
                prods[k][h].append(g * vt)
    return [(_tree_sum(p[0]), _tree_sum(p[1])) for p in prods], gs


_FULL_CHUNK = 25
_UNROLL = 5


@functools.partial(
    pl.kernel,
    out_type=jax.ShapeDtypeStruct((NW, NODE_F32), jnp.float32),
    mesh=_mesh,
    compiler_params=_params,
    scratch_types=[
        pltpu.VMEM((_FULL_CHUNK * NODE_F32,), jnp.float32),
        pltpu.VMEM((_FULL_CHUNK * NODE_F32,), jnp.float32),
        pltpu.VMEM((NODE_F32,), jnp.float32),
        pltpu.VMEM((NODE_F32,), jnp.float32),
        pltpu.SemaphoreType.DMA,
        pltpu.SemaphoreType.DMA,
    ],
)
def _pass_full(u_hbm, vt_hbm, out_hbm, buf0, buf1, vtb, sbuf, sem0, sem1):
    wid = _wid()
    base = wid * (NPW * NODE_F32)
    pltpu.sync_copy(vt_hbm, vtb)
    stride = lax.iota(jnp.int32, 16) * 16
    one = jnp.ones((16,), jnp.float32)
    zero = jnp.zeros((16,), jnp.float32)
    for i in range(OUT):
        sbuf[pl.ds(i * 16, 16)] = zero

    def compute_chunk(buf, ci, _):
        def node_body(ui, _):
            bases = [(ui * _UNROLL + k) * NODE_F32 for k in range(_UNROLL)]
            logits, gs = _group_logits(buf, vtb, bases, stride)
            for k, (l0, l1) in enumerate(logits):
                e0 = jnp.exp(l0)
                e1 = jnp.exp(l1)
                rz = one / _lane_sum_all(e0 + e1)
                c0 = e0 * rz
                c1 = e1 * rz
                for h in range(2):
                    ch = (c0, c1)[h]
                    for f in range(F):
                        plsc.addupdate(
                            sbuf.at[pl.ds((f * 2 + h) * 16, 16)],
                            ch * gs[k][h][f],
                        )
            return 0

        lax.fori_loop(0, _FULL_CHUNK // _UNROLL, node_body, 0)
        return 0

    _double_buffered(
        u_hbm, base, _FULL_CHUNK, NPW // _FULL_CHUNK, buf0, buf1, sem0, sem1,
        compute_chunk, 0,
    )
    pltpu.sync_copy(sbuf, out_hbm.at[wid])


_B_CHUNK = 25


@functools.partial(
    pl.kernel,
    out_type=jax.ShapeDtypeStruct((E,), jnp.float32),
    mesh=_mesh,
    compiler_params=_params,
    scratch_types=[
        pltpu.VMEM((_B_CHUNK * NODE_F32,), jnp.float32),
        pltpu.VMEM((_B_CHUNK * NODE_F32,), jnp.float32),
        pltpu.VMEM((NODE_F32,), jnp.float32),
        pltpu.VMEM((_B_CHUNK * OUT,), jnp.float32),
        pltpu.VMEM((_B_CHUNK * OUT,), jnp.float32),
        pltpu.SemaphoreType.DMA,
        pltpu.SemaphoreType.DMA,
        pltpu.SemaphoreType.DMA,
        pltpu.SemaphoreType.DMA,
    ],
)
def _pass_logits(u_hbm, vt_hbm, b_hbm, buf0, buf1, vtb, bbuf0, bbuf1,
                 sem0, sem1, bsem0, bsem1):
    wid = _wid()
    base = wid * (NPW * NODE_F32)
    bbase = wid * (NPW * OUT)
    pltpu.sync_copy(vt_hbm, vtb)
    stride = lax.iota(jnp.int32, 16) * 16

    def compute_chunk_static(buf, ci, bbuf, bsem):
        def node_body(ui, _):
            bases = [(ui * _UNROLL + k) * NODE_F32 for k in range(_UNROLL)]
            logits, _gs = _group_logits(buf, vtb, bases, stride)
            for k, (l0, l1) in enumerate(logits):
                bbuf[pl.ds((ui * _UNROLL + k) * OUT, 16)] = l0
                bbuf[pl.ds((ui * _UNROLL + k) * OUT + 16, 16)] = l1
            return 0

        lax.fori_loop(0, _B_CHUNK // _UNROLL, node_body, 0)
        dst = b_hbm.at[pl.ds(bbase + ci * (_B_CHUNK * OUT), _B_CHUNK * OUT)]
        sent = pltpu.async_copy(bbuf, dst, bsem)
        return sent

    nchunk = NPW // _B_CHUNK

    pltpu.async_copy(_chunk_src(u_hbm, base, 0, _B_CHUNK), buf0, sem0)

    def pair_body(i, _):
        ci0 = 2 * i
        pltpu.async_copy(_chunk_src(u_hbm, base, ci0 + 1, _B_CHUNK), buf1, sem1)
        pltpu.make_async_copy(
            _chunk_src(u_hbm, base, ci0, _B_CHUNK), buf0, sem0
        ).wait()
        c0 = compute_chunk_static(buf0, ci0, bbuf0, bsem0)
        pltpu.async_copy(_chunk_src(u_hbm, base, ci0 + 2, _B_CHUNK), buf0, sem0)
        pltpu.make_async_copy(
            _chunk_src(u_hbm, base, ci0 + 1, _B_CHUNK), buf1, sem1
        ).wait()
        c1 = compute_chunk_static(buf1, ci0 + 1, bbuf1, bsem1)
        c0.wait()
        c1.wait()
        return 0

    lax.fori_loop(0, nchunk // 2, pair_body, 0)
    pltpu.make_async_copy(
        _chunk_src(u_hbm, base, nchunk - 1, _B_CHUNK), buf0, sem0
    ).wait()
    compute_chunk_static(buf0, nchunk - 1, bbuf0, bsem0).wait()


def _vt(V):
    return V.reshape(2, 16, F).transpose(2, 0, 1).reshape(-1)


def kernel(u_hat, routing_num):
    u_flat = u_hat.reshape(-1)
    sp = _pass_sum(u_flat)
    s0 = sp.reshape(NW, OUT, F).sum(0) / OUT
    v = _squash(s0)

    def body(_, carry):
        V, v = carry
        sp = _pass_full(u_flat, _vt(V))
        s = sp.reshape(NW, F, 2, 16).transpose(0, 2, 3, 1).reshape(NW, OUT, F)
        v2 = _squash(s.sum(0))
        return (V + v2, v2)

    V, v = lax.fori_loop(0, routing_num - 1, body, (v, v))
    b = _pass_logits(u_flat, _vt(V))
    return v, b.reshape(E, 1)


# speedup vs baseline: 1.5446x; 1.5446x over previous
import functools

import jax
import jax.numpy as jnp
from jax import lax
from jax.experimental import pallas as pl
from jax.experimental.pallas import tpu as pltpu
from jax.experimental.pallas import tpu_sc as plsc

IN_NODES = 100000
OUT = 32
F = 16
E = IN_NODES * OUT
NC = 2
NS = 16
NW = NC * NS
NPW = IN_NODES // NW
CHUNK = 25
CE = CHUNK * OUT
NCHUNK = NPW // CHUNK

_mesh = plsc.VectorSubcoreMesh(core_axis_name="c", subcore_axis_name="s")
_params = pltpu.CompilerParams(
    needs_layout_passes=False, use_tc_tiling_on_sc=False
)


def _wid():
    return lax.axis_index("s") * NC + lax.axis_index("c")


def _squash(s):
    sq = jnp.sum(s**2, axis=1, keepdims=True)
    return sq / (1.0 + sq) * (s / jnp.sqrt(sq))


def _tree_sum(ps):
    while len(ps) > 1:
        ps = [a + b for a, b in zip(ps[::2], ps[1::2])] + (
            [ps[-1]] if len(ps) % 2 else []
        )
    return ps[0]


def _lane_shuffle(v, idx):
    return lax.gather(
        v,
        idx[:, None],
        lax.GatherDimensionNumbers(
            offset_dims=(), collapsed_slice_dims=(0,), start_index_map=(0,)
        ),
        (1,),
        mode=lax.GatherScatterMode.PROMISE_IN_BOUNDS,
    )


def _lane_sum_all(v):
    for s in (1, 2, 4, 8):
        idx = jnp.arange(16, dtype=jnp.int32) ^ s
        v = v + _lane_shuffle(v, idx)
    return v


def _copies(u_hbm, ebase, ci, buf, sem):
    return [
        pltpu.make_async_copy(
            u_hbm.at[pl.ds(f * E + ebase + ci * CE, CE)],
            buf.at[pl.ds(f * CE, CE)],
            sem,
        )
        for f in range(F)
    ]


def _double_buffered(u_hbm, ebase, buf0, buf1, sem0, sem1, compute_chunk,
                     init_carry):
    def stage(ci, buf, sem):
        for cp in _copies(u_hbm, ebase, ci, buf, sem):
            cp.start()

    def drain(ci, buf, sem):
        for cp in _copies(u_hbm, ebase, ci, buf, sem):
            cp.wait()

    stage(0, buf0, sem0)

    def pair_body(i, carry):
        ci0 = 2 * i
        stage(ci0 + 1, buf1, sem1)
        drain(ci0, buf0, sem0)
        carry = compute_chunk(buf0, ci0, carry)
        stage(ci0 + 2, buf0, sem0)
        drain(ci0 + 1, buf1, sem1)
        return compute_chunk(buf1, ci0 + 1, carry)

    carry = lax.fori_loop(0, NCHUNK // 2, pair_body, init_carry)
    drain(NCHUNK - 1, buf0, sem0)
    return compute_chunk(buf0, NCHUNK - 1, carry)


@functools.partial(
    pl.kernel,
    out_type=jax.ShapeDtypeStruct((NW, OUT * F), jnp.float32),
    mesh=_mesh,
    compiler_params=_params,
    scratch_types=[
        pltpu.VMEM((F * CE,), jnp.float32),
        pltpu.VMEM((F * CE,), jnp.float32),
        pltpu.VMEM((OUT * F,), jnp.float32),
        pltpu.SemaphoreType.DMA,
        pltpu.SemaphoreType.DMA,
    ],
)
def _pass_sum(u_hbm, out_hbm, buf0, buf1, obuf, sem0, sem1):
    wid = _wid()
    ebase = wid * (NPW * OUT)

    def compute_chunk(buf, ci, accs):
        accs = list(accs)
        for f in range(F):
            for j in range(CE // 16):
                i = f * 2 + (j % 2)
                accs[i] = accs[i] + buf[pl.ds(f * CE + j * 16, 16)]
        return tuple(accs)

    accs = _double_buffered(
        u_hbm, ebase, buf0, buf1, sem0, sem1, compute_chunk,
        tuple(jnp.zeros((16,), jnp.float32) for _ in range(OUT)),
    )
    for i in range(OUT):
        obuf[pl.ds(i * 16, 16)] = accs[i]
    pltpu.sync_copy(obuf, out_hbm.at[wid])


def _node_logits(buf, vtb, ui):
    ps = [[], []]
    for f in range(F):
        for h in range(2):
            u = buf[pl.ds(f * CE + ui * OUT + h * 16, 16)]
            ps[h].append(u * vtb[pl.ds(f * 32 + h * 16, 16)])
    return _tree_sum(ps[0]), _tree_sum(ps[1])


def _softmax2(l0, l1, one):
    e0 = jnp.exp(l0)
    e1 = jnp.exp(l1)
    rz = one / _lane_sum_all(e0 + e1)
    return e0 * rz, e1 * rz


@functools.partial(
    pl.kernel,
    out_type=jax.ShapeDtypeStruct((NW, OUT * F), jnp.float32),
    mesh=_mesh,
    compiler_params=_params,
    scratch_types=[
        pltpu.VMEM((F * CE,), jnp.float32),
        pltpu.VMEM((F * CE,), jnp.float32),
        pltpu.VMEM((OUT * F,), jnp.float32),
        pltpu.VMEM((OUT * F,), jnp.float32),
        pltpu.SemaphoreType.DMA,
        pltpu.SemaphoreType.DMA,
    ],
)
def _pass_full(u_hbm, vt_hbm, out_hbm, buf0, buf1, vtb, obuf, sem0, sem1):
    wid = _wid()
    ebase = wid * (NPW * OUT)
    pltpu.sync_copy(vt_hbm, vtb)
    one = jnp.ones((16,), jnp.float32)

    def compute_chunk(buf, ci, accs):
        def node_body(ui, accs):
            l0, l1 = _node_logits(buf, vtb, ui)
            c0, c1 = _softmax2(l0, l1, one)
            accs = list(accs)
            for f in range(F):
                u0 = buf[pl.ds(f * CE + ui * OUT, 16)]
                u1 = buf[pl.ds(f * CE + ui * OUT + 16, 16)]
                accs[f * 2] = accs[f * 2] + c0 * u0
                accs[f * 2 + 1] = accs[f * 2 + 1] + c1 * u1
            return tuple(accs)

        return lax.fori_loop(0, CHUNK, node_body, accs)

    accs = _double_buffered(
        u_hbm, ebase, buf0, buf1, sem0, sem1, compute_chunk,
        tuple(jnp.zeros((16,), jnp.float32) for _ in range(OUT)),
    )
    for i in range(OUT):
        obuf[pl.ds(i * 16, 16)] = accs[i]
    pltpu.sync_copy(obuf, out_hbm.at[wid])


@functools.partial(
    pl.kernel,
    out_type=jax.ShapeDtypeStruct((E,), jnp.float32),
    mesh=_mesh,
    compiler_params=_params,
    scratch_types=[
        pltpu.VMEM((F * CE,), jnp.float32),
        pltpu.VMEM((F * CE,), jnp.float32),
        pltpu.VMEM((OUT * F,), jnp.float32),
        pltpu.VMEM((CE,), jnp.float32),
        pltpu.VMEM((CE,), jnp.float32),
        pltpu.SemaphoreType.DMA,
        pltpu.SemaphoreType.DMA,
        pltpu.SemaphoreType.DMA,
        pltpu.SemaphoreType.DMA,
    ],
)
def _pass_logits(u_hbm, vt_hbm, b_hbm, buf0, buf1, vtb, bbuf0, bbuf1,
                 sem0, sem1, bsem0, bsem1):
    wid = _wid()
    ebase = wid * (NPW * OUT)
    pltpu.sync_copy(vt_hbm, vtb)

    def compute_chunk(buf, ci, bbuf, bsem):
        def node_body(ui, _):
            l0, l1 = _node_logits(buf, vtb, ui)
            bbuf[pl.ds(ui * OUT, 16)] = l0
            bbuf[pl.ds(ui * OUT + 16, 16)] = l1
            return 0

        lax.fori_loop(0, CHUNK, node_body, 0)
        dst = b_hbm.at[pl.ds(ebase + ci * CE, CE)]
        return pltpu.async_copy(bbuf, dst, bsem)

    def stage(ci, buf, sem):
        for cp in _copies(u_hbm, ebase, ci, buf, sem):
            cp.start()

    def drain(ci, buf, sem):
        for cp in _copies(u_hbm, ebase, ci, buf, sem):
            cp.wait()

    stage(0, buf0, sem0)

    def pair_body(i, _):
        ci0 = 2 * i
        stage(ci0 + 1, buf1, sem1)
        drain(ci0, buf0, sem0)
        w0 = compute_chunk(buf0, ci0, bbuf0, bsem0)
        stage(ci0 + 2, buf0, sem0)
        drain(ci0 + 1, buf1, sem1)
        w1 = compute_chunk(buf1, ci0 + 1, bbuf1, bsem1)
        w0.wait()
        w1.wait()
        return 0

    lax.fori_loop(0, NCHUNK // 2, pair_body, 0)
    drain(NCHUNK - 1, buf0, sem0)
    compute_chunk(buf0, NCHUNK - 1, bbuf0, bsem0).wait()


def _vt(V):
    return V.reshape(2, 16, F).transpose(2, 0, 1).reshape(-1)


def kernel(u_hat, routing_num):
    ut = u_hat.T.reshape(-1)
    sp = _pass_sum(ut)
    s0 = sp.reshape(NW, F, 2, 16).transpose(0, 2, 3, 1).reshape(NW, OUT, F)
    v = _squash(s0.sum(0) / OUT)

    def body(_, carry):
        V, v = carry
        sp = _pass_full(ut, _vt(V))
        s = sp.reshape(NW, F, 2, 16).transpose(0, 2, 3, 1).reshape(NW, OUT, F)
        v2 = _squash(s.sum(0))
        return (V + v2, v2)

    V, v = lax.fori_loop(0, routing_num - 1, body, (v, v))
    b = _pass_logits(ut, _vt(V))
    return v, b.reshape(E, 1)


# speedup vs baseline: 1.9504x; 1.2627x over previous
import functools

import jax
import jax.numpy as jnp
from jax import lax
from jax.experimental import pallas as pl
from jax.experimental.pallas import tpu as pltpu
from jax.experimental.pallas import tpu_sc as plsc

IN_NODES = 100000
OUT = 32
F = 16
E = IN_NODES * OUT
NC = 2
NS = 16
NW = NC * NS
NBLK = E // 128
NBW = NBLK // NW
CB = 11
NCHB = NBW // CB
BLK = 16 * 128

_mesh = plsc.VectorSubcoreMesh(core_axis_name="c", subcore_axis_name="s")
_params = pltpu.CompilerParams(
    needs_layout_passes=False, use_tc_tiling_on_sc=False
)


def _wid():
    return lax.axis_index("s") * NC + lax.axis_index("c")


def _squash(s):
    sq = jnp.sum(s**2, axis=1, keepdims=True)
    return sq / (1.0 + sq) * (s / jnp.sqrt(sq))


def _tree_sum(ps):
    while len(ps) > 1:
        ps = [a + b for a, b in zip(ps[::2], ps[1::2])] + (
            [ps[-1]] if len(ps) % 2 else []
        )
    return ps[0]


def _lane_shuffle(v, idx):
    return lax.gather(
        v,
        idx[:, None],
        lax.GatherDimensionNumbers(
            offset_dims=(), collapsed_slice_dims=(0,), start_index_map=(0,)
        ),
        (1,),
        mode=lax.GatherScatterMode.PROMISE_IN_BOUNDS,
    )


def _lane_sum_all(v):
    for s in (1, 2, 4, 8):
        idx = jnp.arange(16, dtype=jnp.int32) ^ s
        v = v + _lane_shuffle(v, idx)
    return v


def _copies(u_hbm, blk0, nblk, buf, sem):
    return [
        pltpu.make_async_copy(
            u_hbm.at[pl.ds(tf * (NBLK * 1024) + blk0 * 1024, nblk * 1024)],
            buf.at[pl.ds(tf * (nblk * 1024), nblk * 1024)],
            sem,
        )
        for tf in range(2)
    ]


def _double_buffered(u_hbm, wblk0, buf0, buf1, sem0, sem1, compute_chunk,
                     init_carry):
    def stage(ci, buf, sem):
        for cp in _copies(u_hbm, wblk0 + ci * CB, CB, buf, sem):
            cp.start()

    def drain(ci, buf, sem):
        for cp in _copies(u_hbm, wblk0 + ci * CB, CB, buf, sem):
            cp.wait()

    stage(0, buf0, sem0)

    def pair_body(i, carry):
        ci0 = 2 * i
        stage(ci0 + 1, buf1, sem1)
        drain(ci0, buf0, sem0)
        carry = compute_chunk(buf0, ci0, carry)
        stage(ci0 + 2, buf0, sem0)
        drain(ci0 + 1, buf1, sem1)
        return compute_chunk(buf1, ci0 + 1, carry)

    carry = lax.fori_loop(0, NCHB // 2, pair_body, init_carry)
    drain(NCHB - 1, buf0, sem0)
    return compute_chunk(buf0, NCHB - 1, carry)


def _u_vec(buf, nblk, bi, k, f, h):
    tf, fi = f // 8, f % 8
    return buf[
        pl.ds(tf * (nblk * 1024) + bi * 1024 + fi * 128 + k * 32 + h * 16, 16)
    ]


def _node_logits(buf, vtb, nblk, bi, k):
    ps = [[], []]
    for f in range(F):
        for h in range(2):
            u = _u_vec(buf, nblk, bi, k, f, h)
            ps[h].append(u * vtb[pl.ds(f * 32 + h * 16, 16)])
    return _tree_sum(ps[0]), _tree_sum(ps[1])


def _softmax2(l0, l1, one):
    e0 = jnp.exp(l0)
    e1 = jnp.exp(l1)
    rz = one / _lane_sum_all(e0 + e1)
    return e0 * rz, e1 * rz


def _extra_blk(wid):
    eb = (NBW * NW) + lax.rem(wid, 8)
    scale = jnp.where(wid < 8, 1.0, 0.0).astype(jnp.float32)
    return eb, jnp.full((16,), scale)


@functools.partial(
    pl.kernel,
    out_type=jax.ShapeDtypeStruct((NW, OUT * F), jnp.float32),
    mesh=_mesh,
    compiler_params=_params,
    scratch_types=[
        pltpu.VMEM((2 * CB * 1024,), jnp.float32),
        pltpu.VMEM((2 * CB * 1024,), jnp.float32),
        pltpu.VMEM((2 * 1024,), jnp.float32),
        pltpu.VMEM((OUT * F,), jnp.float32),
        pltpu.SemaphoreType.DMA,
        pltpu.SemaphoreType.DMA,
    ],
)
def _pass_sum(u_hbm, out_hbm, buf0, buf1, ebuf, obuf, sem0, sem1):
    wid = _wid()
    wblk0 = wid * NBW

    def blk_sums(buf, nblk, bi):
        out = []
        for f in range(F):
            for h in range(2):
                out.append(
                    _tree_sum([_u_vec(buf, nblk, bi, k, f, h) for k in range(4)])
                )
        return out

    def compute_chunk(buf, ci, accs):
        def blk_body(bi, accs):
            bs = blk_sums(buf, CB, bi)
            return tuple(a + b for a, b in zip(accs, bs))

        return lax.fori_loop(0, CB, blk_body, accs)

    accs = _double_buffered(
        u_hbm, wblk0, buf0, buf1, sem0, sem1, compute_chunk,
        tuple(jnp.zeros((16,), jnp.float32) for _ in range(OUT)),
    )
    eb, scale = _extra_blk(wid)
    for cp in _copies(u_hbm, eb, 1, ebuf, sem0):
        cp.start()
    for cp in _copies(u_hbm, eb, 1, ebuf, sem0):
        cp.wait()
    accs = tuple(
        a + scale * b for a, b in zip(accs, blk_sums(ebuf, 1, 0))
    )
    for i in range(OUT):
        obuf[pl.ds(i * 16, 16)] = accs[i]
    pltpu.sync_copy(obuf, out_hbm.at[wid])


@functools.partial(
    pl.kernel,
    out_type=jax.ShapeDtypeStruct((NW, OUT * F), jnp.float32),
    mesh=_mesh,
    compiler_params=_params,
    scratch_types=[
        pltpu.VMEM((2 * CB * 1024,), jnp.float32),
        pltpu.VMEM((2 * CB * 1024,), jnp.float32),
        pltpu.VMEM((2 * 1024,), jnp.float32),
        pltpu.VMEM((OUT * F,), jnp.float32),
        pltpu.VMEM((OUT * F,), jnp.float32),
        pltpu.SemaphoreType.DMA,
        pltpu.SemaphoreType.DMA,
    ],
)
def _pass_full(u_hbm, vt_hbm, out_hbm, buf0, buf1, ebuf, vtb, obuf, sem0, sem1):
    wid = _wid()
    wblk0 = wid * NBW
    pltpu.sync_copy(vt_hbm, vtb)
    one = jnp.ones((16,), jnp.float32)

    def node_accumulate(buf, nblk, bi, k, accs, scale=None):
        l0, l1 = _node_logits(buf, vtb, nblk, bi, k)
        c0, c1 = _softmax2(l0, l1, one)
        if scale is not None:
            c0 = c0 * scale
            c1 = c1 * scale
        accs = list(accs)
        for f in range(F):
            accs[f * 2] = accs[f * 2] + c0 * _u_vec(buf, nblk, bi, k, f, 0)
            accs[f * 2 + 1] = (
                accs[f * 2 + 1] + c1 * _u_vec(buf, nblk, bi, k, f, 1)
            )
        return tuple(accs)

    def compute_chunk(buf, ci, accs):
        def blk_body(bi, accs):
            for k in range(4):
                accs = node_accumulate(buf, CB, bi, k, accs)
            return accs

        return lax.fori_loop(0, CB, blk_body, accs)

    accs = _double_buffered(
        u_hbm, wblk0, buf0, buf1, sem0, sem1, compute_chunk,
        tuple(jnp.zeros((16,), jnp.float32) for _ in range(OUT)),
    )
    eb, scale = _extra_blk(wid)
    for cp in _copies(u_hbm, eb, 1, ebuf, sem0):
        cp.start()
    for cp in _copies(u_hbm, eb, 1, ebuf, sem0):
        cp.wait()
    for k in range(4):
        accs = node_accumulate(ebuf, 1, 0, k, accs, scale)
    for i in range(OUT):
        obuf[pl.ds(i * 16, 16)] = accs[i]
    pltpu.sync_copy(obuf, out_hbm.at[wid])


@functools.partial(
    pl.kernel,
    out_type=jax.ShapeDtypeStruct((E,), jnp.float32),
    mesh=_mesh,
    compiler_params=_params,
    scratch_types=[
        pltpu.VMEM((2 * CB * 1024,), jnp.float32),
        pltpu.VMEM((2 * CB * 1024,), jnp.float32),
        pltpu.VMEM((2 * 1024,), jnp.float32),
        pltpu.VMEM((OUT * F,), jnp.float32),
        pltpu.VMEM((CB * 128,), jnp.float32),
        pltpu.VMEM((CB * 128,), jnp.float32),
        pltpu.SemaphoreType.DMA,
        pltpu.SemaphoreType.DMA,
        pltpu.SemaphoreType.DMA,
        pltpu.SemaphoreType.DMA,
    ],
)
def _pass_logits(u_hbm, vt_hbm, b_hbm, buf0, buf1, ebuf, vtb, bbuf0, bbuf1,
                 sem0, sem1, bsem0, bsem1):
    wid = _wid()
    wblk0 = wid * NBW
    pltpu.sync_copy(vt_hbm, vtb)

    def compute_chunk(buf, ci, bbuf, bsem):
        def blk_body(bi, _):
            for k in range(4):
                l0, l1 = _node_logits(buf, vtb, CB, bi, k)
                bbuf[pl.ds(bi * 128 + k * 32, 16)] = l0
                bbuf[pl.ds(bi * 128 + k * 32 + 16, 16)] = l1
            return 0

        lax.fori_loop(0, CB, blk_body, 0)
        dst = b_hbm.at[pl.ds((wblk0 + ci * CB) * 128, CB * 128)]
        return pltpu.async_copy(bbuf, dst, bsem)

    def stage(ci, buf, sem):
        for cp in _copies(u_hbm, wblk0 + ci * CB, CB, buf, sem):
            cp.start()

    def drain(ci, buf, sem):
        for cp in _copies(u_hbm, wblk0 + ci * CB, CB, buf, sem):
            cp.wait()

    stage(0, buf0, sem0)

    def pair_body(i, _):
        ci0 = 2 * i
        stage(ci0 + 1, buf1, sem1)
        drain(ci0, buf0, sem0)
        w0 = compute_chunk(buf0, ci0, bbuf0, bsem0)
        stage(ci0 + 2, buf0, sem0)
        drain(ci0 + 1, buf1, sem1)
        w1 = compute_chunk(buf1, ci0 + 1, bbuf1, bsem1)
        w0.wait()
        w1.wait()
        return 0

    lax.fori_loop(0, NCHB // 2, pair_body, 0)
    drain(NCHB - 1, buf0, sem0)
    compute_chunk(buf0, NCHB - 1, bbuf0, bsem0).wait()

    eb, _ = _extra_blk(wid)
    for cp in _copies(u_hbm, eb, 1, ebuf, sem0):
        cp.start()
    for cp in _copies(u_hbm, eb, 1, ebuf, sem0):
        cp.wait()
    for k in range(4):
        l0, l1 = _node_logits(ebuf, vtb, 1, 0, k)
        bbuf0[pl.ds(k * 32, 16)] = l0
        bbuf0[pl.ds(k * 32 + 16, 16)] = l1
    pltpu.async_copy(
        bbuf0.at[pl.ds(0, 128)], b_hbm.at[pl.ds(eb * 128, 128)], bsem0
    ).wait()


def _vt(V):
    return V.reshape(2, 16, F).transpose(2, 0, 1).reshape(-1)


def kernel(u_hat, routing_num):
    ut = (
        u_hat.reshape(NBLK, 128, 2, 8).transpose(2, 0, 3, 1).reshape(-1)
    )
    sp = _pass_sum(ut)
    s0 = sp.reshape(NW, F, 2, 16).transpose(0, 2, 3, 1).reshape(NW, OUT, F)
    v = _squash(s0.sum(0) / OUT)

    def body(_, carry):
        V, v = carry
        sp = _pass_full(ut, _vt(V))
        s = sp.reshape(NW, F, 2, 16).transpose(0, 2, 3, 1).reshape(NW, OUT, F)
        v2 = _squash(s.sum(0))
        return (V + v2, v2)

    V, v = lax.fori_loop(0, routing_num - 1, body, (v, v))
    b = _pass_logits(ut, _vt(V))
    return v, b.reshape(E, 1)


# speedup vs baseline: 2.9184x; 1.4963x over previous
import functools

import jax
import jax.numpy as jnp
from jax import lax
from jax.experimental import pallas as pl
from jax.experimental.pallas import tpu as pltpu
from jax.experimental.pallas import tpu_sc as plsc

IN_NODES = 100000
OUT = 32
F = 16
E = IN_NODES * OUT
NC = 2
NS = 16
NW = NC * NS
NBLK = E // 128
NBW = NBLK // NW
CB = 11
NCHB = NBW // CB
BLK = 16 * 128

_mesh = plsc.VectorSubcoreMesh(core_axis_name="c", subcore_axis_name="s")
_params = pltpu.CompilerParams(
    needs_layout_passes=False, use_tc_tiling_on_sc=False
)


def _wid():
    return lax.axis_index("s") * NC + lax.axis_index("c")


def _squash(s):
    sq = jnp.sum(s**2, axis=1, keepdims=True)
    return sq / (1.0 + sq) * (s / jnp.sqrt(sq))


def _tree_sum(ps):
    while len(ps) > 1:
        ps = [a + b for a, b in zip(ps[::2], ps[1::2])] + (
            [ps[-1]] if len(ps) % 2 else []
        )
    return ps[0]


def _lane_shuffle(v, idx):
    return lax.gather(
        v,
        idx[:, None],
        lax.GatherDimensionNumbers(
            offset_dims=(), collapsed_slice_dims=(0,), start_index_map=(0,)
        ),
        (1,),
        mode=lax.GatherScatterMode.PROMISE_IN_BOUNDS,
    )


def _lane_sum_all(v):
    for s in (1, 2, 4, 8):
        idx = jnp.arange(16, dtype=jnp.int32) ^ s
        v = v + _lane_shuffle(v, idx)
    return v


def _copies(u_hbm, blk0, nblk, buf, sem):
    return [
        pltpu.make_async_copy(
            u_hbm.at[pl.ds(tf * (NBLK * 1024) + blk0 * 1024, nblk * 1024)],
            buf.at[pl.ds(tf * (nblk * 1024), nblk * 1024)],
            sem,
        )
        for tf in range(2)
    ]


def _double_buffered(u_hbm, wblk0, buf0, buf1, sem0, sem1, compute_chunk,
                     init_carry):
    def stage(ci, buf, sem):
        for cp in _copies(u_hbm, wblk0 + ci * CB, CB, buf, sem):
            cp.start()

    def drain(ci, buf, sem):
        for cp in _copies(u_hbm, wblk0 + ci * CB, CB, buf, sem):
            cp.wait()

    stage(0, buf0, sem0)

    def pair_body(i, carry):
        ci0 = 2 * i
        stage(ci0 + 1, buf1, sem1)
        drain(ci0, buf0, sem0)
        carry = compute_chunk(buf0, ci0, carry)
        stage(ci0 + 2, buf0, sem0)
        drain(ci0 + 1, buf1, sem1)
        return compute_chunk(buf1, ci0 + 1, carry)

    carry = lax.fori_loop(0, NCHB // 2, pair_body, init_carry)
    drain(NCHB - 1, buf0, sem0)
    return compute_chunk(buf0, NCHB - 1, carry)


def _u_vec(buf, nblk, bi, k, f, h):
    tf, fi = f // 8, f % 8
    return buf[
        pl.ds(tf * (nblk * 1024) + bi * 1024 + fi * 128 + k * 32 + h * 16, 16)
    ]


def _block_logits(buf, vtb, nblk, bi):
    ls = [[None, None] for _ in range(4)]
    for f in range(F):
        for h in range(2):
            vt = vtb[pl.ds(f * 32 + h * 16, 16)]
            for k in range(4):
                p = _u_vec(buf, nblk, bi, k, f, h) * vt
                ls[k][h] = p if ls[k][h] is None else ls[k][h] + p
    return ls


def _softmax2(l0, l1, one):
    e0 = jnp.exp(l0)
    e1 = jnp.exp(l1)
    rz = one / _lane_sum_all(e0 + e1)
    return e0 * rz, e1 * rz


def _extra_blk(wid):
    eb = (NBW * NW) + lax.rem(wid, 8)
    scale = jnp.where(wid < 8, 1.0, 0.0).astype(jnp.float32)
    return eb, jnp.full((16,), scale)


@functools.partial(
    pl.kernel,
    out_type=jax.ShapeDtypeStruct((NW, OUT * F), jnp.float32),
    mesh=_mesh,
    compiler_params=_params,
    scratch_types=[
        pltpu.VMEM((2 * CB * 1024,), jnp.float32),
        pltpu.VMEM((2 * CB * 1024,), jnp.float32),
        pltpu.VMEM((2 * 1024,), jnp.float32),
        pltpu.VMEM((OUT * F,), jnp.float32),
        pltpu.SemaphoreType.DMA,
        pltpu.SemaphoreType.DMA,
    ],
)
def _pass_sum(u_hbm, out_hbm, buf0, buf1, ebuf, obuf, sem0, sem1):
    wid = _wid()
    wblk0 = wid * NBW

    def blk_sums(buf, nblk, bi):
        out = []
        for f in range(F):
            for h in range(2):
                out.append(
                    _tree_sum([_u_vec(buf, nblk, bi, k, f, h) for k in range(4)])
                )
        return out

    def compute_chunk(buf, ci, accs):
        def blk_body(bi, accs):
            bs = blk_sums(buf, CB, bi)
            return tuple(a + b for a, b in zip(accs, bs))

        return lax.fori_loop(0, CB, blk_body, accs)

    accs = _double_buffered(
        u_hbm, wblk0, buf0, buf1, sem0, sem1, compute_chunk,
        tuple(jnp.zeros((16,), jnp.float32) for _ in range(OUT)),
    )
    eb, scale = _extra_blk(wid)
    for cp in _copies(u_hbm, eb, 1, ebuf, sem0):
        cp.start()
    for cp in _copies(u_hbm, eb, 1, ebuf, sem0):
        cp.wait()
    accs = tuple(
        a + scale * b for a, b in zip(accs, blk_sums(ebuf, 1, 0))
    )
    for i in range(OUT):
        obuf[pl.ds(i * 16, 16)] = accs[i]
    pltpu.sync_copy(obuf, out_hbm.at[wid])


@functools.partial(
    pl.kernel,
    out_type=jax.ShapeDtypeStruct((NW, OUT * F), jnp.float32),
    mesh=_mesh,
    compiler_params=_params,
    scratch_types=[
        pltpu.VMEM((2 * CB * 1024,), jnp.float32),
        pltpu.VMEM((2 * CB * 1024,), jnp.float32),
        pltpu.VMEM((2 * 1024,), jnp.float32),
        pltpu.VMEM((OUT * F,), jnp.float32),
        pltpu.VMEM((OUT * F,), jnp.float32),
        pltpu.SemaphoreType.DMA,
        pltpu.SemaphoreType.DMA,
    ],
)
def _pass_full(u_hbm, vt_hbm, out_hbm, buf0, buf1, ebuf, vtb, obuf, sem0, sem1):
    wid = _wid()
    wblk0 = wid * NBW
    pltpu.sync_copy(vt_hbm, vtb)
    one = jnp.ones((16,), jnp.float32)

    def block_accumulate(buf, nblk, bi, accs, scale=None):
        ls = _block_logits(buf, vtb, nblk, bi)
        accs = list(accs)
        for k in range(4):
            c0, c1 = _softmax2(ls[k][0], ls[k][1], one)
            if scale is not None:
                c0 = c0 * scale
                c1 = c1 * scale
            for f in range(F):
                accs[f * 2] = accs[f * 2] + c0 * _u_vec(buf, nblk, bi, k, f, 0)
                accs[f * 2 + 1] = (
                    accs[f * 2 + 1] + c1 * _u_vec(buf, nblk, bi, k, f, 1)
                )
        return tuple(accs)

    def compute_chunk(buf, ci, accs):
        def blk_body(bi, accs):
            return block_accumulate(buf, CB, bi, accs)

        return lax.fori_loop(0, CB, blk_body, accs)

    accs = _double_buffered(
        u_hbm, wblk0, buf0, buf1, sem0, sem1, compute_chunk,
        tuple(jnp.zeros((16,), jnp.float32) for _ in range(OUT)),
    )
    eb, scale = _extra_blk(wid)
    for cp in _copies(u_hbm, eb, 1, ebuf, sem0):
        cp.start()
    for cp in _copies(u_hbm, eb, 1, ebuf, sem0):
        cp.wait()
    accs = block_accumulate(ebuf, 1, 0, accs, scale)
    for i in range(OUT):
        obuf[pl.ds(i * 16, 16)] = accs[i]
    pltpu.sync_copy(obuf, out_hbm.at[wid])


@functools.partial(
    pl.kernel,
    out_type=jax.ShapeDtypeStruct((E,), jnp.float32),
    mesh=_mesh,
    compiler_params=_params,
    scratch_types=[
        pltpu.VMEM((2 * CB * 1024,), jnp.float32),
        pltpu.VMEM((2 * CB * 1024,), jnp.float32),
        pltpu.VMEM((2 * 1024,), jnp.float32),
        pltpu.VMEM((OUT * F,), jnp.float32),
        pltpu.VMEM((CB * 128,), jnp.float32),
        pltpu.VMEM((CB * 128,), jnp.float32),
        pltpu.SemaphoreType.DMA,
        pltpu.SemaphoreType.DMA,
        pltpu.SemaphoreType.DMA,
        pltpu.SemaphoreType.DMA,
    ],
)
def _pass_logits(u_hbm, vt_hbm, b_hbm, buf0, buf1, ebuf, vtb, bbuf0, bbuf1,
                 sem0, sem1, bsem0, bsem1):
    wid = _wid()
    wblk0 = wid * NBW
    pltpu.sync_copy(vt_hbm, vtb)

    def compute_chunk(buf, ci, bbuf, bsem):
        def blk_body(bi, _):
            ls = _block_logits(buf, vtb, CB, bi)
            for k in range(4):
                bbuf[pl.ds(bi * 128 + k * 32, 16)] = ls[k][0]
                bbuf[pl.ds(bi * 128 + k * 32 + 16, 16)] = ls[k][1]
            return 0

        lax.fori_loop(0, CB, blk_body, 0)
        dst = b_hbm.at[pl.ds((wblk0 + ci * CB) * 128, CB * 128)]
        return pltpu.async_copy(bbuf, dst, bsem)

    def stage(ci, buf, sem):
        for cp in _copies(u_hbm, wblk0 + ci * CB, CB, buf, sem):
            cp.start()

    def drain(ci, buf, sem):
        for cp in _copies(u_hbm, wblk0 + ci * CB, CB, buf, sem):
            cp.wait()

    stage(0, buf0, sem0)

    def pair_body(i, _):
        ci0 = 2 * i
        stage(ci0 + 1, buf1, sem1)
        drain(ci0, buf0, sem0)
        w0 = compute_chunk(buf0, ci0, bbuf0, bsem0)
        stage(ci0 + 2, buf0, sem0)
        drain(ci0 + 1, buf1, sem1)
        w1 = compute_chunk(buf1, ci0 + 1, bbuf1, bsem1)
        w0.wait()
        w1.wait()
        return 0

    lax.fori_loop(0, NCHB // 2, pair_body, 0)
    drain(NCHB - 1, buf0, sem0)
    compute_chunk(buf0, NCHB - 1, bbuf0, bsem0).wait()

    eb, _ = _extra_blk(wid)
    for cp in _copies(u_hbm, eb, 1, ebuf, sem0):
        cp.start()
    for cp in _copies(u_hbm, eb, 1, ebuf, sem0):
        cp.wait()
    els = _block_logits(ebuf, vtb, 1, 0)
    for k in range(4):
        bbuf0[pl.ds(k * 32, 16)] = els[k][0]
        bbuf0[pl.ds(k * 32 + 16, 16)] = els[k][1]
    pltpu.async_copy(
        bbuf0.at[pl.ds(0, 128)], b_hbm.at[pl.ds(eb * 128, 128)], bsem0
    ).wait()


def _vt(V):
    return V.reshape(2, 16, F).transpose(2, 0, 1).reshape(-1)


def kernel(u_hat, routing_num):
    ut = (
        u_hat.reshape(NBLK, 128, 2, 8).transpose(2, 0, 3, 1).reshape(-1)
    )
    sp = _pass_sum(ut)
    s0 = sp.reshape(NW, F, 2, 16).transpose(0, 2, 3, 1).reshape(NW, OUT, F)
    v = _squash(s0.sum(0) / OUT)

    def body(_, carry):
        V, v = carry
        sp = _pass_full(ut, _vt(V))
        s = sp.reshape(NW, F, 2, 16).transpose(0, 2, 3, 1).reshape(NW, OUT, F)
        v2 = _squash(s.sum(0))
        return (V + v2, v2)

    V, v = lax.fori_loop(0, routing_num - 1, body, (v, v))
    b = _pass_logits(ut, _vt(V))
    return v, b.reshape(E, 1)


# speedup vs baseline: 2.9199x; 1.0005x over previous
import functools

import jax
import jax.numpy as jnp
from jax import lax
from jax.experimental import pallas as pl
from jax.experimental.pallas import tpu as pltpu
from jax.experimental.pallas import tpu_sc as plsc

IN_NODES = 100000
OUT = 32
F = 16
E = IN_NODES * OUT
NC = 2
NS = 16
NW = NC * NS
NBLK = E // 128
NBW = NBLK // NW
CB = 11
NCHB = NBW // CB

_mesh = plsc.VectorSubcoreMesh(core_axis_name="c", subcore_axis_name="s")
_params = pltpu.CompilerParams(
    needs_layout_passes=False, use_tc_tiling_on_sc=False
)


def _wid():
    return lax.axis_index("s") * NC + lax.axis_index("c")


def _squash(s):
    sq = jnp.sum(s**2, axis=1, keepdims=True)
    return sq / (1.0 + sq) * (s / jnp.sqrt(sq))


def _tree_sum(ps):
    while len(ps) > 1:
        ps = [a + b for a, b in zip(ps[::2], ps[1::2])] + (
            [ps[-1]] if len(ps) % 2 else []
        )
    return ps[0]


def _lane_shuffle(v, idx):
    return lax.gather(
        v,
        idx[:, None],
        lax.GatherDimensionNumbers(
            offset_dims=(), collapsed_slice_dims=(0,), start_index_map=(0,)
        ),
        (1,),
        mode=lax.GatherScatterMode.PROMISE_IN_BOUNDS,
    )


def _lane_sum_all(v):
    for s in (1, 2, 4, 8):
        idx = jnp.arange(16, dtype=jnp.int32) ^ s
        v = v + _lane_shuffle(v, idx)
    return v


def _copies(u_hbm, blk0, nblk, buf, sem):
    return [
        pltpu.make_async_copy(
            u_hbm.at[pl.ds(tf * (NBLK * 1024) + blk0 * 1024, nblk * 1024)],
            buf.at[pl.ds(tf * (nblk * 1024), nblk * 1024)],
            sem,
        )
        for tf in range(2)
    ]


def _double_buffered(u_hbm, wblk0, buf0, buf1, sem0, sem1, compute_chunk,
                     init_carry):
    def stage(ci, buf, sem):
        for cp in _copies(u_hbm, wblk0 + ci * CB, CB, buf, sem):
            cp.start()

    def drain(ci, buf, sem):
        for cp in _copies(u_hbm, wblk0 + ci * CB, CB, buf, sem):
            cp.wait()

    stage(0, buf0, sem0)

    def pair_body(i, carry):
        ci0 = 2 * i
        stage(ci0 + 1, buf1, sem1)
        drain(ci0, buf0, sem0)
        carry = compute_chunk(buf0, ci0, carry)
        stage(ci0 + 2, buf0, sem0)
        drain(ci0 + 1, buf1, sem1)
        return compute_chunk(buf1, ci0 + 1, carry)

    carry = lax.fori_loop(0, NCHB // 2, pair_body, init_carry)
    drain(NCHB - 1, buf0, sem0)
    return compute_chunk(buf0, NCHB - 1, carry)


def _u_vec(buf, nblk, bi, k, f, h):
    tf, fi = f // 8, f % 8
    return buf[
        pl.ds(tf * (nblk * 1024) + bi * 1024 + fi * 128 + k * 32 + h * 16, 16)
    ]


def _block_logits(buf, vtb, nblk, bi):
    ls = [[None, None] for _ in range(4)]
    for f in range(F):
        for h in range(2):
            vt = vtb[pl.ds(f * 32 + h * 16, 16)]
            for k in range(4):
                p = _u_vec(buf, nblk, bi, k, f, h) * vt
                ls[k][h] = p if ls[k][h] is None else ls[k][h] + p
    return ls


def _softmax2(l0, l1, one):
    e0 = jnp.exp(l0)
    e1 = jnp.exp(l1)
    rz = one / _lane_sum_all(e0 + e1)
    return e0 * rz, e1 * rz


def _extra_blk(wid):
    eb = (NBW * NW) + lax.rem(wid, 8)
    scale = jnp.where(wid < 8, 1.0, 0.0).astype(jnp.float32)
    return eb, jnp.full((16,), scale)


@functools.partial(
    pl.kernel,
    out_type=jax.ShapeDtypeStruct((NW, OUT * F), jnp.float32),
    mesh=_mesh,
    compiler_params=_params,
    scratch_types=[
        pltpu.VMEM((2 * CB * 1024,), jnp.float32),
        pltpu.VMEM((2 * CB * 1024,), jnp.float32),
        pltpu.VMEM((2 * 1024,), jnp.float32),
        pltpu.VMEM((OUT * F,), jnp.float32),
        pltpu.SemaphoreType.DMA,
        pltpu.SemaphoreType.DMA,
    ],
)
def _pass_sum(u_hbm, out_hbm, buf0, buf1, ebuf, obuf, sem0, sem1):
    wid = _wid()
    wblk0 = wid * NBW

    def blk_sums(buf, nblk, bi):
        out = []
        for f in range(F):
            for h in range(2):
                out.append(
                    _tree_sum([_u_vec(buf, nblk, bi, k, f, h) for k in range(4)])
                )
        return out

    def compute_chunk(buf, ci, accs):
        def blk_body(bi, accs):
            bs = blk_sums(buf, CB, bi)
            return tuple(a + b for a, b in zip(accs, bs))

        return lax.fori_loop(0, CB, blk_body, accs)

    accs = _double_buffered(
        u_hbm, wblk0, buf0, buf1, sem0, sem1, compute_chunk,
        tuple(jnp.zeros((16,), jnp.float32) for _ in range(OUT)),
    )
    eb, scale = _extra_blk(wid)
    for cp in _copies(u_hbm, eb, 1, ebuf, sem0):
        cp.start()
    for cp in _copies(u_hbm, eb, 1, ebuf, sem0):
        cp.wait()
    accs = tuple(
        a + scale * b for a, b in zip(accs, blk_sums(ebuf, 1, 0))
    )
    for i in range(OUT):
        obuf[pl.ds(i * 16, 16)] = accs[i]
    pltpu.sync_copy(obuf, out_hbm.at[wid])


@functools.partial(
    pl.kernel,
    out_type=jax.ShapeDtypeStruct((NW, OUT * F), jnp.float32),
    mesh=_mesh,
    compiler_params=_params,
    scratch_types=[
        pltpu.VMEM((2 * CB * 1024,), jnp.float32),
        pltpu.VMEM((2 * CB * 1024,), jnp.float32),
        pltpu.VMEM((2 * 1024,), jnp.float32),
        pltpu.VMEM((OUT * F,), jnp.float32),
        pltpu.VMEM((OUT * F,), jnp.float32),
        pltpu.SemaphoreType.DMA,
        pltpu.SemaphoreType.DMA,
    ],
)
def _pass_full(u_hbm, vt_hbm, out_hbm, buf0, buf1, ebuf, vtb, obuf, sem0, sem1):
    wid = _wid()
    wblk0 = wid * NBW
    pltpu.sync_copy(vt_hbm, vtb)
    one = jnp.ones((16,), jnp.float32)

    def block_accumulate(buf, nblk, bi, accs, scale=None):
        ls = _block_logits(buf, vtb, nblk, bi)
        accs = list(accs)
        for k in range(4):
            c0, c1 = _softmax2(ls[k][0], ls[k][1], one)
            if scale is not None:
                c0 = c0 * scale
                c1 = c1 * scale
            for f in range(F):
                accs[f * 2] = accs[f * 2] + c0 * _u_vec(buf, nblk, bi, k, f, 0)
                accs[f * 2 + 1] = (
                    accs[f * 2 + 1] + c1 * _u_vec(buf, nblk, bi, k, f, 1)
                )
        return tuple(accs)

    def compute_chunk(buf, ci, accs):
        def blk_body(bi, accs):
            return block_accumulate(buf, CB, bi, accs)

        return lax.fori_loop(0, CB, blk_body, accs)

    accs = _double_buffered(
        u_hbm, wblk0, buf0, buf1, sem0, sem1, compute_chunk,
        tuple(jnp.zeros((16,), jnp.float32) for _ in range(OUT)),
    )
    eb, scale = _extra_blk(wid)
    for cp in _copies(u_hbm, eb, 1, ebuf, sem0):
        cp.start()
    for cp in _copies(u_hbm, eb, 1, ebuf, sem0):
        cp.wait()
    accs = block_accumulate(ebuf, 1, 0, accs, scale)
    for i in range(OUT):
        obuf[pl.ds(i * 16, 16)] = accs[i]
    pltpu.sync_copy(obuf, out_hbm.at[wid])


@functools.partial(
    pl.kernel,
    out_type=jax.ShapeDtypeStruct((E,), jnp.float32),
    mesh=_mesh,
    compiler_params=_params,
    scratch_types=[
        pltpu.VMEM((2 * CB * 1024,), jnp.float32),
        pltpu.VMEM((2 * CB * 1024,), jnp.float32),
        pltpu.VMEM((2 * 1024,), jnp.float32),
        pltpu.VMEM((OUT * F,), jnp.float32),
        pltpu.VMEM((CB * 128,), jnp.float32),
        pltpu.VMEM((CB * 128,), jnp.float32),
        pltpu.SemaphoreType.DMA,
        pltpu.SemaphoreType.DMA,
        pltpu.SemaphoreType.DMA,
        pltpu.SemaphoreType.DMA,
    ],
)
def _pass_logits(u_hbm, vt_hbm, b_hbm, buf0, buf1, ebuf, vtb, bbuf0, bbuf1,
                 sem0, sem1, bsem0, bsem1):
    wid = _wid()
    wblk0 = wid * NBW
    pltpu.sync_copy(vt_hbm, vtb)

    def compute_chunk(buf, ci, bbuf, bsem):
        def blk_body(bi, _):
            ls = _block_logits(buf, vtb, CB, bi)
            for k in range(4):
                bbuf[pl.ds(bi * 128 + k * 32, 16)] = ls[k][0]
                bbuf[pl.ds(bi * 128 + k * 32 + 16, 16)] = ls[k][1]
            return 0

        lax.fori_loop(0, CB, blk_body, 0)
        dst = b_hbm.at[pl.ds((wblk0 + ci * CB) * 128, CB * 128)]
        return pltpu.async_copy(bbuf, dst, bsem)

    def stage(ci, buf, sem):
        for cp in _copies(u_hbm, wblk0 + ci * CB, CB, buf, sem):
            cp.start()

    def drain(ci, buf, sem):
        for cp in _copies(u_hbm, wblk0 + ci * CB, CB, buf, sem):
            cp.wait()

    stage(0, buf0, sem0)

    def pair_body(i, _):
        ci0 = 2 * i
        stage(ci0 + 1, buf1, sem1)
        drain(ci0, buf0, sem0)
        w0 = compute_chunk(buf0, ci0, bbuf0, bsem0)
        stage(ci0 + 2, buf0, sem0)
        drain(ci0 + 1, buf1, sem1)
        w1 = compute_chunk(buf1, ci0 + 1, bbuf1, bsem1)
        w0.wait()
        w1.wait()
        return 0

    lax.fori_loop(0, NCHB // 2, pair_body, 0)
    drain(NCHB - 1, buf0, sem0)
    compute_chunk(buf0, NCHB - 1, bbuf0, bsem0).wait()

    eb, _ = _extra_blk(wid)
    for cp in _copies(u_hbm, eb, 1, ebuf, sem0):
        cp.start()
    for cp in _copies(u_hbm, eb, 1, ebuf, sem0):
        cp.wait()
    els = _block_logits(ebuf, vtb, 1, 0)
    for k in range(4):
        bbuf0[pl.ds(k * 32, 16)] = els[k][0]
        bbuf0[pl.ds(k * 32 + 16, 16)] = els[k][1]
    pltpu.async_copy(
        bbuf0.at[pl.ds(0, 128)], b_hbm.at[pl.ds(eb * 128, 128)], bsem0
    ).wait()


def _vt(V):
    return V.reshape(2, 16, F).transpose(2, 0, 1).reshape(-1)


def kernel(u_hat, routing_num):
    ut = (
        u_hat.reshape(NBLK, 128, 2, 8).transpose(2, 0, 3, 1).reshape(-1)
    )
    sp = _pass_sum(ut)
    s0 = sp.reshape(NW, F, 2, 16).transpose(0, 2, 3, 1).reshape(NW, OUT, F)
    v = _squash(s0.sum(0) / OUT)

    def body(_, carry):
        V, v = carry
        sp = _pass_full(ut, _vt(V))
        s = sp.reshape(NW, F, 2, 16).transpose(0, 2, 3, 1).reshape(NW, OUT, F)
        v2 = _squash(s.sum(0))
        return (V + v2, v2)

    V, v = lax.fori_loop(0, routing_num - 1, body, (v, v))
    b = _pass_logits(ut, _vt(V))
    return v, b.reshape(E, 1)
